# baseline (speedup 1.0000x reference)
import jax
import jax.numpy as jnp
from jax.experimental import pallas as pl

N_VALS = 10000
N_CONS = 10000
H = 128


def _mlp(x, W1, b1, W2, b2):
    return jax.nn.relu(x @ W1 + b1) @ W2 + b2


def _conv(x_src, x_dst, ei, ea, p, n_dst):
    e = ea @ p['We'] + p['be']
    m = jax.nn.relu(x_src[ei[0]] + e)
    agg = jax.ops.segment_sum(m, ei[1], num_segments=n_dst)
    return jax.nn.relu(agg @ p['Wa'] + x_dst @ p['Wr'] + p['br'])


def _pred_head_kernel(v_ref, w1_ref, b1_ref, w2_ref, b2_ref, w3_ref, b3_ref, o_ref):
    h = jnp.maximum(v_ref[...] @ w1_ref[...] + b1_ref[...], 0.0)
    h = jnp.maximum(h @ w2_ref[...] + b2_ref[...], 0.0)
    o_ref[...] = h @ w3_ref[...] + b3_ref[...]


def _pred_head(vals, pr):
    BLK = 1000
    return pl.pallas_call(
        _pred_head_kernel,
        grid=(N_VALS // BLK,),
        in_specs=[
            pl.BlockSpec((BLK, H), lambda i: (i, 0)),
            pl.BlockSpec((H, H), lambda i: (0, 0)),
            pl.BlockSpec((H,), lambda i: (0,)),
            pl.BlockSpec((H, H), lambda i: (0, 0)),
            pl.BlockSpec((H,), lambda i: (0,)),
            pl.BlockSpec((H, 1), lambda i: (0, 0)),
            pl.BlockSpec((1,), lambda i: (0,)),
        ],
        out_specs=pl.BlockSpec((BLK, 1), lambda i: (i, 0)),
        out_shape=jax.ShapeDtypeStruct((N_VALS, 1), jnp.float32),
    )(vals, pr['W1'], pr['b1'], pr['W2'], pr['b2'], pr['W3'], pr['b3'])


def kernel(b, q, x_start, edge_attr_vc, edge_attr_cv, params, edge_index_vc, edge_index_cv):
    enc = params['enc']
    cons = _mlp(b[:, None], *enc['b'])
    vals = _mlp(x_start[:, None], *enc['s']) + _mlp(q[:, None], *enc['q'])
    for lp in params['layers']:
        cons = _conv(vals, cons, edge_index_vc, edge_attr_vc, lp['vc'], N_CONS)
        vals = _conv(cons, vals, edge_index_cv, edge_attr_cv, lp['cv'], N_VALS)
    out = _pred_head(vals, params['pred'])
    return out.squeeze(-1)


# speedup vs baseline: 1.9260x; 1.9260x over previous
import functools

import jax
import jax.numpy as jnp
from jax import lax
from jax.experimental import pallas as pl
from jax.experimental.pallas import tpu as pltpu
from jax.experimental.pallas import tpu_sc as plsc

N_VALS = 10000
N_CONS = 10000
NDST = 10000
E = 320000
H = 128

NC = 2
NS = 16
NW = NC * NS
EPW = E // NW
KB = 128
NG = 79
EPWP = NG * KB
NDSTP = 10112
RPS = NDSTP // NS

BLK = 1000


def _conv_body(table, src3, dst3, ea3, we, out, src_v, dst_v, ea_v, rows_v,
               we_v, acc, sem):
    cidx = lax.axis_index("c")
    sid = lax.axis_index("s")
    wid = sid * NC + cidx

    pltpu.sync_copy(we, we_v)
    pltpu.sync_copy(src3.at[wid], src_v)
    pltpu.sync_copy(dst3.at[wid], dst_v)
    pltpu.sync_copy(ea3.at[wid], ea_v)

    def zero_body(r, _):
        for cc in range(8):
            rows_v[r, pl.ds(cc * 16, 16)] = jnp.zeros((16,), jnp.float32)
        return 0
    lax.fori_loop(0, KB, zero_body, 0)
    for j in range(RPS // KB):
        pltpu.sync_copy(rows_v, acc.at[pl.ds(sid * RPS + j * KB, KB)])
    rem = RPS % KB
    if rem:
        pltpu.sync_copy(rows_v.at[pl.ds(0, rem)],
                        acc.at[pl.ds(sid * RPS + (RPS // KB) * KB, rem)])
    plsc.subcore_barrier()

    def chunk_body(g, _):
        pltpu.async_copy(table.at[src_v.at[g]], rows_v, sem).wait()

        def grp_body(gg, _):
            base = gg * 16
            ea16 = ea_v[g, pl.ds(base, 16)]
            for j in range(16):
                av = jnp.broadcast_to(ea16[j], (16,))
                for cc in range(8):
                    sl = pl.ds(cc * 16, 16)
                    rows_v[base + j, sl] = jnp.maximum(
                        rows_v[base + j, sl] + av * we_v[sl], 0.0)
            return 0
        lax.fori_loop(0, KB // 16, grp_body, 0)
        pltpu.sync_copy(rows_v, acc.at[dst_v.at[g]], add=True)
        return 0
    lax.fori_loop(0, NG, chunk_body, 0)

    plsc.subcore_barrier()
    pltpu.sync_copy(acc.at[pl.ds(sid * RPS, RPS)],
                    out.at[cidx, pl.ds(sid * RPS, RPS)])


def _conv_sc(table_be, src3, dst3, ea3, we_row):
    run = pl.kernel(
        _conv_body,
        mesh=plsc.VectorSubcoreMesh(core_axis_name="c", subcore_axis_name="s"),
        out_type=jax.ShapeDtypeStruct((NC, NDSTP, H), jnp.float32),
        scratch_types=[
            pltpu.VMEM((NG, KB), jnp.int32),
            pltpu.VMEM((NG, KB), jnp.int32),
            pltpu.VMEM((NG, KB), jnp.float32),
            pltpu.VMEM((KB, H), jnp.float32),
            pltpu.VMEM((H,), jnp.float32),
            pltpu.VMEM_SHARED((NDSTP, H), jnp.float32),
            pltpu.SemaphoreType.DMA,
        ],
    )
    return run(table_be, src3, dst3, ea3, we_row)


def _enc_kernel(b_ref, q_ref, x_ref,
                wb1, bb1, wb2, bb2, ws1, bs1, ws2, bs2, wq1, bq1, wq2, bq2,
                bevc, cons_ref, vals_ref, valsbe_ref):
    bb = b_ref[...]
    cons_ref[...] = (jnp.maximum(bb * wb1[...] + bb1[...], 0.0)
                     @ wb2[...] + bb2[...])
    xx = x_ref[...]
    qq = q_ref[...]
    vals = (jnp.maximum(xx * ws1[...] + bs1[...], 0.0) @ ws2[...]
            + bs2[...]
            + jnp.maximum(qq * wq1[...] + bq1[...], 0.0) @ wq2[...]
            + bq2[...])
    vals_ref[...] = vals
    valsbe_ref[...] = vals + bevc[...]


def _encode(b, q, x_start, enc, be_vc):
    w_spec = pl.BlockSpec((1, H), lambda i: (0, 0))
    b_spec = pl.BlockSpec((H,), lambda i: (0,))
    m_spec = pl.BlockSpec((H, H), lambda i: (0, 0))
    v_spec = pl.BlockSpec((BLK, 1), lambda i: (i, 0))
    o_spec = pl.BlockSpec((BLK, H), lambda i: (i, 0))
    (wb1, bb1, wb2, bb2) = enc['b']
    (ws1, bs1, ws2, bs2) = enc['s']
    (wq1, bq1, wq2, bq2) = enc['q']
    return pl.pallas_call(
        _enc_kernel,
        grid=(NDST // BLK,),
        in_specs=[v_spec, v_spec, v_spec,
                  w_spec, b_spec, m_spec, b_spec,
                  w_spec, b_spec, m_spec, b_spec,
                  w_spec, b_spec, m_spec, b_spec,
                  b_spec],
        out_specs=[o_spec, o_spec, o_spec],
        out_shape=[jax.ShapeDtypeStruct((N_CONS, H), jnp.float32),
                   jax.ShapeDtypeStruct((N_VALS, H), jnp.float32),
                   jax.ShapeDtypeStruct((N_VALS, H), jnp.float32)],
    )(b[:, None], q[:, None], x_start[:, None],
      wb1, bb1, wb2, bb2, ws1, bs1, ws2, bs2, wq1, bq1, wq2, bq2, be_vc)


def _combine_kernel(p_ref, xd_ref, wa, wr, br, benext, out_ref, outbe_ref):
    agg = p_ref[0] + p_ref[1]
    out = jnp.maximum(agg @ wa[...] + xd_ref[...] @ wr[...] + br[...], 0.0)
    out_ref[...] = out
    outbe_ref[...] = out + benext[...]


def _combine_be(partials, x_dst, cp, be_next):
    b_spec = pl.BlockSpec((H,), lambda i: (0,))
    m_spec = pl.BlockSpec((H, H), lambda i: (0, 0))
    o_spec = pl.BlockSpec((BLK, H), lambda i: (i, 0))
    return pl.pallas_call(
        _combine_kernel,
        grid=(NDST // BLK,),
        in_specs=[pl.BlockSpec((NC, BLK, H), lambda i: (0, i, 0)),
                  o_spec, m_spec, m_spec, b_spec, b_spec],
        out_specs=[o_spec, o_spec],
        out_shape=[jax.ShapeDtypeStruct((NDST, H), jnp.float32),
                   jax.ShapeDtypeStruct((NDST, H), jnp.float32)],
    )(partials, x_dst, cp['Wa'], cp['Wr'], cp['br'], be_next)


def _combine_pred_kernel(p_ref, xd_ref, wa, wr, br, w1, b1, w2, b2, w3,
                         out_ref):
    agg = p_ref[0] + p_ref[1]
    v = jnp.maximum(agg @ wa[...] + xd_ref[...] @ wr[...] + br[...], 0.0)
    h = jnp.maximum(v @ w1[...] + b1[...], 0.0)
    h = jnp.maximum(h @ w2[...] + b2[...], 0.0)
    out_ref[...] = h @ w3[...]


def _combine_pred(partials, x_dst, cp, pr):
    b_spec = pl.BlockSpec((H,), lambda i: (0,))
    m_spec = pl.BlockSpec((H, H), lambda i: (0, 0))
    return pl.pallas_call(
        _combine_pred_kernel,
        grid=(N_VALS // BLK,),
        in_specs=[pl.BlockSpec((NC, BLK, H), lambda i: (0, i, 0)),
                  pl.BlockSpec((BLK, H), lambda i: (i, 0)),
                  m_spec, m_spec, b_spec,
                  m_spec, b_spec, m_spec, b_spec,
                  pl.BlockSpec((H, 1), lambda i: (0, 0))],
        out_specs=pl.BlockSpec((BLK, 1), lambda i: (i, 0)),
        out_shape=jax.ShapeDtypeStruct((N_VALS, 1), jnp.float32),
    )(partials, x_dst, cp['Wa'], cp['Wr'], cp['br'],
      pr['W1'], pr['b1'], pr['W2'], pr['b2'], pr['W3'])


def kernel(b, q, x_start, edge_attr_vc, edge_attr_cv, params,
           edge_index_vc, edge_index_cv):
    layers = params['layers']
    pr = params['pred']
    n_layers = len(layers)

    def prep(ei, ea):
        pad = ((0, 0), (0, EPWP - EPW))
        src = jnp.pad(ei[0].reshape(NW, EPW), pad)
        dst = jnp.pad(ei[1].reshape(NW, EPW), pad, constant_values=NDST)
        eav = jnp.pad(ea[:, 0].reshape(NW, EPW), pad)
        return (src.reshape(NW, NG, KB), dst.reshape(NW, NG, KB),
                eav.reshape(NW, NG, KB))

    src_vc, dst_vc, ea_vc = prep(edge_index_vc, edge_attr_vc)
    src_cv, dst_cv, ea_cv = prep(edge_index_cv, edge_attr_cv)

    cons, vals, valsbe = _encode(b, q, x_start, params['enc'],
                                 layers[0]['vc']['be'])

    out = None
    for l, lp in enumerate(layers):
        pvc = _conv_sc(valsbe, src_vc, dst_vc, ea_vc, lp['vc']['We'][0])
        cons, consbe = _combine_be(pvc, cons, lp['vc'], lp['cv']['be'])
        pcv = _conv_sc(consbe, src_cv, dst_cv, ea_cv, lp['cv']['We'][0])
        if l + 1 < n_layers:
            vals, valsbe = _combine_be(pcv, vals, lp['cv'],
                                       layers[l + 1]['vc']['be'])
        else:
            out = _combine_pred(pcv, vals, lp['cv'], pr)

    return out[:, 0] + pr['b3'][0]
